# Initial kernel scaffold; baseline (speedup 1.0000x reference)
import jax
import jax.numpy as jnp
from jax.experimental import pallas as pl


def kernel(feat_paper, feat_author, lin_w0, lin_b0, ln_g0, ln_b0, lin_w1, lin_b1, ln_g1, ln_b1, basis1, comp1, att_q1, att_k1, bias1, basis2, comp2, att_q2, att_k2, bias2, tr_g, tr_b, tr_w, tr_bias, edge_index, edge_type, forward_type):
    raise NotImplementedError("write your pallas kernel here")


# speedup vs baseline: 2.4129x; 2.4129x over previous
import functools

import jax
import jax.numpy as jnp
from jax import lax
from jax.experimental import pallas as pl

N1, N2 = 6000, 4000
N = N1 + N2
E = 320000
IN, HID, OUT = 128, 128, 64
R, NB = 8, 30


def _prolog_body(f_ref, w_ref, b_ref, g_ref, bb_ref, o_ref):
    h = jnp.dot(f_ref[...], w_ref[...], preferred_element_type=jnp.float32)
    h = jnp.maximum(h + b_ref[...], 0.0)
    m = jnp.mean(h, axis=-1, keepdims=True)
    v = jnp.mean((h - m) ** 2, axis=-1, keepdims=True)
    o_ref[...] = (h - m) / jnp.sqrt(v + 1e-5) * g_ref[...] + bb_ref[...]


def _prolog(feat, w, b, g, bb, blk):
    n = feat.shape[0]
    return pl.pallas_call(
        _prolog_body,
        grid=(n // blk,),
        in_specs=[
            pl.BlockSpec((blk, IN), lambda i: (i, 0)),
            pl.BlockSpec((IN, IN), lambda i: (0, 0)),
            pl.BlockSpec((1, IN), lambda i: (0, 0)),
            pl.BlockSpec((1, IN), lambda i: (0, 0)),
            pl.BlockSpec((1, IN), lambda i: (0, 0)),
        ],
        out_specs=pl.BlockSpec((blk, IN), lambda i: (i, 0)),
        out_shape=jax.ShapeDtypeStruct((n, IN), jnp.float32),
    )(feat, w, b.reshape(1, IN), g.reshape(1, IN), bb.reshape(1, IN))


def _xw_body(comp_ref, basis_ref, x_ref, qk_ref, xw_ref, tab_ref):
    r = pl.program_id(0)
    comp_row = comp_ref[pl.ds(r, 1), :]
    wr = jnp.dot(comp_row, basis_ref[...].reshape(NB, HID * HID),
                 preferred_element_type=jnp.float32).reshape(HID, HID)
    xw = jnp.dot(x_ref[...], wr, preferred_element_type=jnp.float32)
    xw_ref[0] = xw
    tab_ref[0] = jnp.dot(xw, qk_ref[...], preferred_element_type=jnp.float32)


def _xw_tables(x, comp, basis, q, k):
    qk = jnp.concatenate([q, k], axis=1)
    return pl.pallas_call(
        _xw_body,
        grid=(R,),
        in_specs=[
            pl.BlockSpec((R, NB), lambda r: (0, 0)),
            pl.BlockSpec((NB, HID, HID), lambda r: (0, 0, 0)),
            pl.BlockSpec((N, HID), lambda r: (0, 0)),
            pl.BlockSpec((HID, 2), lambda r: (0, 0)),
        ],
        out_specs=[
            pl.BlockSpec((1, N, HID), lambda r: (r, 0, 0)),
            pl.BlockSpec((1, N, 2), lambda r: (r, 0, 0)),
        ],
        out_shape=[
            jax.ShapeDtypeStruct((R, N, HID), jnp.float32),
            jax.ShapeDtypeStruct((R, N, 2), jnp.float32),
        ],
    )(comp, basis, x, qk)


def _epilog_body(x_ref, g_ref, b_ref, w_ref, bias_ref, o_ref):
    x = x_ref[...]
    m = jnp.mean(x, axis=-1, keepdims=True)
    v = jnp.mean((x - m) ** 2, axis=-1, keepdims=True)
    xn = (x - m) / jnp.sqrt(v + 1e-5) * g_ref[...] + b_ref[...]
    o_ref[...] = jnp.dot(xn, w_ref[...], preferred_element_type=jnp.float32) + bias_ref[...]


def _epilog(x, g, b, w, bias):
    return pl.pallas_call(
        _epilog_body,
        grid=(1,),
        in_specs=[
            pl.BlockSpec((N, HID), lambda i: (0, 0)),
            pl.BlockSpec((1, HID), lambda i: (0, 0)),
            pl.BlockSpec((1, HID), lambda i: (0, 0)),
            pl.BlockSpec((HID, OUT), lambda i: (0, 0)),
            pl.BlockSpec((1, OUT), lambda i: (0, 0)),
        ],
        out_specs=pl.BlockSpec((N, OUT), lambda i: (0, 0)),
        out_shape=jax.ShapeDtypeStruct((N, OUT), jnp.float32),
    )(x, g.reshape(1, HID), b.reshape(1, HID), w, bias.reshape(1, OUT))


def _edge_phase(xw, tab, eidx, didx, dst, bias):
    xw_flat = xw.reshape(R * N, HID)
    qi_flat = tab[..., 0].reshape(R * N)
    kj_flat = tab[..., 1].reshape(R * N)
    alpha = jax.nn.leaky_relu(qi_flat[didx] + kj_flat[eidx], 0.2)
    ex = jnp.exp(alpha)
    den = jax.ops.segment_sum(ex, dst, num_segments=N)
    msg = xw_flat[eidx] * ex[:, None]
    acc = jax.ops.segment_sum(msg, dst, num_segments=N)
    return acc / (den[:, None] + 1e-16) + bias


def kernel(feat_paper, feat_author, lin_w0, lin_b0, ln_g0, ln_b0, lin_w1,
           lin_b1, ln_g1, ln_b1, basis1, comp1, att_q1, att_k1, bias1,
           basis2, comp2, att_q2, att_k2, bias2, tr_g, tr_b, tr_w, tr_bias,
           edge_index, edge_type, forward_type):
    h0 = _prolog(feat_paper, lin_w0, lin_b0, ln_g0, ln_b0, 1000)
    h1 = _prolog(feat_author, lin_w1, lin_b1, ln_g1, ln_b1, 1000)
    x = jnp.concatenate([h0, h1], axis=0)

    src = edge_index[0]
    dst = edge_index[1]
    eidx = edge_type * N + src
    didx = edge_type * N + dst

    xw1, tab1 = _xw_tables(x, comp1, basis1, att_q1, att_k1)
    x = _edge_phase(xw1, tab1, eidx, didx, dst, bias1)

    xw2, tab2 = _xw_tables(x, comp2, basis2, att_q2, att_k2)
    x = _edge_phase(xw2, tab2, eidx, didx, dst, bias2)

    return _epilog(x, tr_g, tr_b, tr_w, tr_bias)


# speedup vs baseline: 28.2315x; 11.7001x over previous
import functools

import jax
import jax.numpy as jnp
from jax import lax
from jax.experimental import pallas as pl
from jax.experimental.pallas import tpu as pltpu
from jax.experimental.pallas import tpu_sc as plsc

N1, N2 = 6000, 4000
N = N1 + N2
E = 320000
IN, HID, OUT = 128, 128, 64
R, NB = 8, 30

NW = 32
EW = E // NW
C = 80
NCH = EW // C
WT = 10
WR = N // WT
ZR = 200
NDP = 16384
TD = NDP // 16


def _prolog_body(f_ref, w_ref, b_ref, g_ref, bb_ref, o_ref):
    h = jnp.dot(f_ref[...], w_ref[...], preferred_element_type=jnp.float32)
    h = jnp.maximum(h + b_ref[...], 0.0)
    m = jnp.mean(h, axis=-1, keepdims=True)
    v = jnp.mean((h - m) ** 2, axis=-1, keepdims=True)
    o_ref[...] = (h - m) / jnp.sqrt(v + 1e-5) * g_ref[...] + bb_ref[...]


def _prolog(feat, w, b, g, bb, blk):
    n = feat.shape[0]
    return pl.pallas_call(
        _prolog_body,
        grid=(n // blk,),
        in_specs=[
            pl.BlockSpec((blk, IN), lambda i: (i, 0)),
            pl.BlockSpec((IN, IN), lambda i: (0, 0)),
            pl.BlockSpec((1, IN), lambda i: (0, 0)),
            pl.BlockSpec((1, IN), lambda i: (0, 0)),
            pl.BlockSpec((1, IN), lambda i: (0, 0)),
        ],
        out_specs=pl.BlockSpec((blk, IN), lambda i: (i, 0)),
        out_shape=jax.ShapeDtypeStruct((n, IN), jnp.float32),
    )(feat, w, b.reshape(1, IN), g.reshape(1, IN), bb.reshape(1, IN))


def _xw_body(comp_ref, basis_ref, x_ref, qk_ref, xw_ref, tab_ref):
    r = pl.program_id(0)
    comp_row = comp_ref[pl.ds(r, 1), :]
    wr = jnp.dot(comp_row, basis_ref[...].reshape(NB, HID * HID),
                 preferred_element_type=jnp.float32).reshape(HID, HID)
    xw = jnp.dot(x_ref[...], wr, preferred_element_type=jnp.float32)
    xw_ref[0] = xw
    tab_ref[0] = jnp.dot(xw, qk_ref[...], preferred_element_type=jnp.float32)


def _xw_tables(x, comp, basis, q, k):
    qk = jnp.concatenate([q, k], axis=1)
    return pl.pallas_call(
        _xw_body,
        grid=(R,),
        in_specs=[
            pl.BlockSpec((R, NB), lambda r: (0, 0)),
            pl.BlockSpec((NB, HID, HID), lambda r: (0, 0, 0)),
            pl.BlockSpec((N, HID), lambda r: (0, 0)),
            pl.BlockSpec((HID, 2), lambda r: (0, 0)),
        ],
        out_specs=[
            pl.BlockSpec((1, N, HID), lambda r: (r, 0, 0)),
            pl.BlockSpec((1, N, 2), lambda r: (r, 0, 0)),
        ],
        out_shape=[
            jax.ShapeDtypeStruct((R, N, HID), jnp.float32),
            jax.ShapeDtypeStruct((R, N, 2), jnp.float32),
        ],
    )(comp, basis, x, qk)


def _epilog_body(x_ref, g_ref, b_ref, w_ref, bias_ref, o_ref):
    x = x_ref[...]
    m = jnp.mean(x, axis=-1, keepdims=True)
    v = jnp.mean((x - m) ** 2, axis=-1, keepdims=True)
    xn = (x - m) / jnp.sqrt(v + 1e-5) * g_ref[...] + b_ref[...]
    o_ref[...] = jnp.dot(xn, w_ref[...], preferred_element_type=jnp.float32) + bias_ref[...]


def _epilog(x, g, b, w, bias):
    return pl.pallas_call(
        _epilog_body,
        grid=(1,),
        in_specs=[
            pl.BlockSpec((N, HID), lambda i: (0, 0)),
            pl.BlockSpec((1, HID), lambda i: (0, 0)),
            pl.BlockSpec((1, HID), lambda i: (0, 0)),
            pl.BlockSpec((HID, OUT), lambda i: (0, 0)),
            pl.BlockSpec((1, OUT), lambda i: (0, 0)),
        ],
        out_specs=pl.BlockSpec((N, OUT), lambda i: (0, 0)),
        out_shape=jax.ShapeDtypeStruct((N, OUT), jnp.float32),
    )(x, g.reshape(1, HID), b.reshape(1, HID), w, bias.reshape(1, OUT))


_SC_MESH = plsc.VectorSubcoreMesh(core_axis_name="c", subcore_axis_name="s")


def _edge_sc_body(xw_hbm, qi_hbm, kj_hbm, eidx_hbm, didx_hbm, dst_hbm,
                  acc_out, den0_out, den1_out,
                  eidx_v, didx_v, dst_v, qi_v, kj_v, ex_v, rows_v, zbuf,
                  zden, acc_sh, den_sh, sem_q, sem_k, sem_r):
    cid = lax.axis_index("c")
    sid = lax.axis_index("s")
    wid = sid * 2 + cid

    z16 = jnp.zeros((16,), jnp.float32)

    def zb(i, _):
        for j in range(HID // 16):
            zbuf[i, pl.ds(j * 16, 16)] = z16
        return 0

    lax.fori_loop(0, ZR, zb, 0)
    for j in range(TD // 16):
        zden[pl.ds(j * 16, 16)] = z16
    @pl.when(sid < WT)
    def _zero_acc():
        for t in range(WR // ZR):
            pltpu.sync_copy(zbuf, acc_sh.at[pl.ds(sid * WR + t * ZR, ZR)])
    pltpu.sync_copy(zden, den_sh.at[pl.ds(sid * TD, TD)])
    plsc.subcore_barrier()

    def body(g, _):
        base = wid * EW + g * C
        pltpu.sync_copy(eidx_hbm.at[pl.ds(base, C)], eidx_v)
        pltpu.sync_copy(didx_hbm.at[pl.ds(base, C)], didx_v)
        pltpu.sync_copy(dst_hbm.at[pl.ds(base, C)], dst_v)
        cp_q = pltpu.async_copy(qi_hbm.at[didx_v], qi_v, sem_q)
        cp_k = pltpu.async_copy(kj_hbm.at[eidx_v], kj_v, sem_k)
        cp_r = pltpu.async_copy(xw_hbm.at[eidx_v], rows_v, sem_r)
        cp_q.wait()
        cp_k.wait()
        for j in range(C // 16):
            sl = pl.ds(j * 16, 16)
            a = qi_v[sl] + kj_v[sl]
            a = jnp.where(a >= 0.0, a, 0.2 * a)
            ex_v[sl] = jnp.exp(a)
        cp_r.wait()

        def scale(g2, _):
            ex16 = ex_v[pl.ds(g2 * 16, 16)]
            base_i = g2 * 16
            for l in range(16):
                s = ex16[l]
                for j in range(HID // 16):
                    sl = pl.ds(j * 16, 16)
                    rows_v[base_i + l, sl] = rows_v[base_i + l, sl] * s
            return 0

        lax.fori_loop(0, C // 16, scale, 0)
        pltpu.sync_copy(rows_v, acc_sh.at[dst_v], add=True)
        pltpu.sync_copy(ex_v, den_sh.at[dst_v], add=True)
        return 0

    lax.fori_loop(0, NCH, body, 0)

    plsc.subcore_barrier()
    @pl.when(sid < WT)
    def _wb_acc():
        pltpu.sync_copy(acc_sh.at[pl.ds(sid * WR, WR)],
                        acc_out.at[cid, pl.ds(sid * WR, WR)])
    @pl.when(cid == 0)
    def _wb_den0():
        pltpu.sync_copy(den_sh.at[pl.ds(sid * TD, TD)],
                        den0_out.at[pl.ds(sid * TD, TD)])
    @pl.when(cid == 1)
    def _wb_den1():
        pltpu.sync_copy(den_sh.at[pl.ds(sid * TD, TD)],
                        den1_out.at[pl.ds(sid * TD, TD)])


_edge_sc = functools.partial(
    pl.kernel,
    out_type=(jax.ShapeDtypeStruct((2, N, HID), jnp.float32),
              jax.ShapeDtypeStruct((NDP,), jnp.float32),
              jax.ShapeDtypeStruct((NDP,), jnp.float32)),
    mesh=_SC_MESH,
    scratch_types=[
        pltpu.VMEM((C,), jnp.int32),
        pltpu.VMEM((C,), jnp.int32),
        pltpu.VMEM((C,), jnp.int32),
        pltpu.VMEM((C,), jnp.float32),
        pltpu.VMEM((C,), jnp.float32),
        pltpu.VMEM((C,), jnp.float32),
        pltpu.VMEM((C, HID), jnp.float32),
        pltpu.VMEM((ZR, HID), jnp.float32),
        pltpu.VMEM((TD,), jnp.float32),
        pltpu.VMEM_SHARED((N, HID), jnp.float32),
        pltpu.VMEM_SHARED((NDP,), jnp.float32),
        pltpu.SemaphoreType.DMA,
        pltpu.SemaphoreType.DMA,
        pltpu.SemaphoreType.DMA,
    ],
)(_edge_sc_body)


def _combine_body(a_ref, d0_ref, d1_ref, b_ref, o_ref):
    den = d0_ref[...] + d1_ref[...]
    o_ref[...] = (a_ref[0] + a_ref[1]) / (den + 1e-16) + b_ref[...]


def _combine(acc, den0, den1, bias, blk=1000):
    return pl.pallas_call(
        _combine_body,
        grid=(N // blk,),
        in_specs=[
            pl.BlockSpec((2, blk, HID), lambda i: (0, i, 0)),
            pl.BlockSpec((blk, 1), lambda i: (i, 0)),
            pl.BlockSpec((blk, 1), lambda i: (i, 0)),
            pl.BlockSpec((1, HID), lambda i: (0, 0)),
        ],
        out_specs=pl.BlockSpec((blk, HID), lambda i: (i, 0)),
        out_shape=jax.ShapeDtypeStruct((N, HID), jnp.float32),
    )(acc, den0[:N].reshape(N, 1), den1[:N].reshape(N, 1), bias.reshape(1, HID))


def _edge_phase(xw, tab, eidx, didx, dst, bias):
    xw_flat = xw.reshape(R * N, HID)
    qi_flat = tab[..., 0].reshape(R * N)
    kj_flat = tab[..., 1].reshape(R * N)
    acc, den0, den1 = _edge_sc(xw_flat, qi_flat, kj_flat, eidx, didx, dst)
    return _combine(acc, den0, den1, bias)


def kernel(feat_paper, feat_author, lin_w0, lin_b0, ln_g0, ln_b0, lin_w1,
           lin_b1, ln_g1, ln_b1, basis1, comp1, att_q1, att_k1, bias1,
           basis2, comp2, att_q2, att_k2, bias2, tr_g, tr_b, tr_w, tr_bias,
           edge_index, edge_type, forward_type):
    h0 = _prolog(feat_paper, lin_w0, lin_b0, ln_g0, ln_b0, 1000)
    h1 = _prolog(feat_author, lin_w1, lin_b1, ln_g1, ln_b1, 1000)
    x = jnp.concatenate([h0, h1], axis=0)

    src = edge_index[0]
    dst = edge_index[1]
    eidx = edge_type * N + src
    didx = edge_type * N + dst

    xw1, tab1 = _xw_tables(x, comp1, basis1, att_q1, att_k1)
    x = _edge_phase(xw1, tab1, eidx, didx, dst, bias1)

    xw2, tab2 = _xw_tables(x, comp2, basis2, att_q2, att_k2)
    x = _edge_phase(xw2, tab2, eidx, didx, dst, bias2)

    return _epilog(x, tr_g, tr_b, tr_w, tr_bias)


# speedup vs baseline: 68.6725x; 2.4325x over previous
import functools

import jax
import jax.numpy as jnp
from jax import lax
from jax.experimental import pallas as pl
from jax.experimental.pallas import tpu as pltpu
from jax.experimental.pallas import tpu_sc as plsc

N1, N2 = 6000, 4000
N = N1 + N2
E = 320000
IN, HID, OUT = 128, 128, 64
R, NB = 8, 30

NW = 32
EW = E // NW
C = 64
PADW = 240
EWP = EW + PADW
NCH = EWP // C
NB5 = 5
WT = 10
WR = N // WT
ACCP = 10048
NDP = 10240
TD = NDP // 16
ZR = 40


def _prolog_body(f_ref, w_ref, b_ref, g_ref, bb_ref, o_ref):
    h = jnp.dot(f_ref[...], w_ref[...], preferred_element_type=jnp.float32)
    h = jnp.maximum(h + b_ref[...], 0.0)
    m = jnp.mean(h, axis=-1, keepdims=True)
    v = jnp.mean((h - m) ** 2, axis=-1, keepdims=True)
    o_ref[...] = (h - m) / jnp.sqrt(v + 1e-5) * g_ref[...] + bb_ref[...]


def _prolog(feat, w, b, g, bb, blk):
    n = feat.shape[0]
    return pl.pallas_call(
        _prolog_body,
        grid=(n // blk,),
        in_specs=[
            pl.BlockSpec((blk, IN), lambda i: (i, 0)),
            pl.BlockSpec((IN, IN), lambda i: (0, 0)),
            pl.BlockSpec((1, IN), lambda i: (0, 0)),
            pl.BlockSpec((1, IN), lambda i: (0, 0)),
            pl.BlockSpec((1, IN), lambda i: (0, 0)),
        ],
        out_specs=pl.BlockSpec((blk, IN), lambda i: (i, 0)),
        out_shape=jax.ShapeDtypeStruct((n, IN), jnp.float32),
    )(feat, w, b.reshape(1, IN), g.reshape(1, IN), bb.reshape(1, IN))


def _xw_body(comp_ref, basis_ref, x_ref, xw_ref):
    r = pl.program_id(0)
    comp_row = comp_ref[pl.ds(r, 1), :]
    wr = jnp.dot(comp_row, basis_ref[...].reshape(NB, HID * HID),
                 preferred_element_type=jnp.float32).reshape(HID, HID)
    xw_ref[...] = jnp.dot(x_ref[...], wr, preferred_element_type=jnp.float32)


def _xw_tables(x, comp, basis):
    return pl.pallas_call(
        _xw_body,
        grid=(R,),
        in_specs=[
            pl.BlockSpec((R, NB), lambda r: (0, 0)),
            pl.BlockSpec((NB, HID, HID), lambda r: (0, 0, 0)),
            pl.BlockSpec((N, HID), lambda r: (0, 0)),
        ],
        out_specs=pl.BlockSpec((N, HID), lambda r: (r, 0)),
        out_shape=jax.ShapeDtypeStruct((R * N, HID), jnp.float32),
    )(comp, basis, x)


def _tabs_body(qk_ref, x_ref, o_ref):
    o_ref[...] = lax.dot_general(
        qk_ref[...], x_ref[...], (((1,), (1,)), ((), ())),
        preferred_element_type=jnp.float32)


def _tabs(qkmat, x):
    return pl.pallas_call(
        _tabs_body,
        grid=(1,),
        in_specs=[
            pl.BlockSpec((2 * R, HID), lambda i: (0, 0)),
            pl.BlockSpec((N, HID), lambda i: (0, 0)),
        ],
        out_specs=pl.BlockSpec((2 * R, N), lambda i: (0, 0)),
        out_shape=jax.ShapeDtypeStruct((2 * R, N), jnp.float32),
    )(qkmat, x)


def _epilog_body(x_ref, g_ref, b_ref, w_ref, bias_ref, o_ref):
    x = x_ref[...]
    m = jnp.mean(x, axis=-1, keepdims=True)
    v = jnp.mean((x - m) ** 2, axis=-1, keepdims=True)
    xn = (x - m) / jnp.sqrt(v + 1e-5) * g_ref[...] + b_ref[...]
    o_ref[...] = jnp.dot(xn, w_ref[...], preferred_element_type=jnp.float32) + bias_ref[...]


def _epilog(x, g, b, w, bias):
    return pl.pallas_call(
        _epilog_body,
        grid=(1,),
        in_specs=[
            pl.BlockSpec((N, HID), lambda i: (0, 0)),
            pl.BlockSpec((1, HID), lambda i: (0, 0)),
            pl.BlockSpec((1, HID), lambda i: (0, 0)),
            pl.BlockSpec((HID, OUT), lambda i: (0, 0)),
            pl.BlockSpec((1, OUT), lambda i: (0, 0)),
        ],
        out_specs=pl.BlockSpec((N, OUT), lambda i: (0, 0)),
        out_shape=jax.ShapeDtypeStruct((N, OUT), jnp.float32),
    )(x, g.reshape(1, HID), b.reshape(1, HID), w, bias.reshape(1, OUT))


_SC_MESH = plsc.VectorSubcoreMesh(core_axis_name="c", subcore_axis_name="s")


def _edge_sc_body(xw_hbm, qi_hbm, kj_hbm, eidx_hbm, pidx_hbm,
                  acc_out, den0_out, den1_out,
                  pidx5, eidx5, didx5, dst5, qi5, kj5, ex5, rows_g,
                  zden, acc_sh, den_sh, sem_idx, sem_dat, sem_out):
    cid = lax.axis_index("c")
    sid = lax.axis_index("s")
    wid = sid * 2 + cid

    z16 = jnp.zeros((16,), jnp.float32)

    def zb(i, _):
        for j in range(HID // 16):
            rows_g[0, i, pl.ds(j * 16, 16)] = z16
        return 0

    lax.fori_loop(0, ZR, zb, 0)
    for j in range(TD // 16):
        zden[pl.ds(j * 16, 16)] = z16

    @pl.when(sid < WT)
    def _zero_acc():
        for t in range(WR // ZR):
            pltpu.sync_copy(rows_g.at[0, pl.ds(0, ZR)],
                            acc_sh.at[pl.ds(sid * WR + t * ZR, ZR)])

    pltpu.sync_copy(zden, den_sh.at[pl.ds(sid * TD, TD)])
    plsc.subcore_barrier()

    def issue_idx(g, b):
        base = wid * EWP + g * C
        pltpu.async_copy(eidx_hbm.at[pl.ds(base, C)], eidx5.at[b],
                         sem_idx.at[b])
        pltpu.async_copy(pidx_hbm.at[pl.ds(base, C)], pidx5.at[b],
                         sem_idx.at[b])

    def wait_idx(g, b):
        base = wid * EWP + g * C
        pltpu.make_async_copy(eidx_hbm.at[pl.ds(base, C)], eidx5.at[b],
                              sem_idx.at[b]).wait()
        pltpu.make_async_copy(pidx_hbm.at[pl.ds(base, C)], pidx5.at[b],
                              sem_idx.at[b]).wait()

    def unpack_idx(b):
        for j in range(C // 16):
            sl = pl.ds(j * 16, 16)
            v = pidx5[b, sl]
            didx5[b, sl] = v & 0x1FFFF
            dst5[b, sl] = v >> 17

    def issue_dat(b):
        pltpu.async_copy(qi_hbm.at[didx5.at[b]], qi5.at[b], sem_dat.at[b])
        pltpu.async_copy(kj_hbm.at[eidx5.at[b]], kj5.at[b], sem_dat.at[b])
        pltpu.async_copy(xw_hbm.at[eidx5.at[b]], rows_g.at[b], sem_dat.at[b])

    def wait_qk(b):
        pltpu.make_async_copy(qi_hbm.at[didx5.at[b]], qi5.at[b],
                              sem_dat.at[b]).wait()
        pltpu.make_async_copy(kj_hbm.at[eidx5.at[b]], kj5.at[b],
                              sem_dat.at[b]).wait()

    def wait_rows(b):
        pltpu.make_async_copy(xw_hbm.at[eidx5.at[b]], rows_g.at[b],
                              sem_dat.at[b]).wait()

    def issue_out(g, b):
        pltpu.async_copy(rows_g.at[b], acc_sh.at[dst5.at[b]],
                         sem_out.at[b], add=True)
        pltpu.async_copy(ex5.at[b], den_sh.at[dst5.at[b]],
                         sem_out.at[b], add=True)

    def wait_out(g, b):
        pltpu.make_async_copy(rows_g.at[b], acc_sh.at[dst5.at[b]],
                              sem_out.at[b]).wait()
        pltpu.make_async_copy(ex5.at[b], den_sh.at[dst5.at[b]],
                              sem_out.at[b]).wait()

    for gg in range(NB5):
        issue_idx(gg, gg)
    for gg in range(3):
        wait_idx(gg, gg)
        unpack_idx(gg)
        issue_dat(gg)

    def outer(o, _):
        for b in range(NB5):
            g = o * NB5 + b
            b3 = (b + 3) % NB5

            @pl.when(g >= 2)
            def _drain():
                wait_out(g - 2, b3)

            wait_qk(b)
            for j in range(C // 16):
                sl = pl.ds(j * 16, 16)
                a = qi5[b, sl] + kj5[b, sl]
                a = jnp.where(a >= 0.0, a, 0.2 * a)
                ex5[b, sl] = jnp.exp(a)
            @pl.when(g + 3 < NCH)
            def _next_dat():
                wait_idx(g + 3, b3)
                unpack_idx(b3)
                issue_dat(b3)

            wait_rows(b)

            def scale_grp(g2, _, b=b):
                ex16 = ex5[b, pl.ds(g2 * 16, 16)]
                for l in range(16):
                    s = ex16[l]
                    row = g2 * 16 + l
                    for j in range(HID // 16):
                        sl = pl.ds(j * 16, 16)
                        rows_g[b, row, sl] = rows_g[b, row, sl] * s
                return 0

            lax.fori_loop(0, C // 16, scale_grp, 0)
            issue_out(g, b)

            @pl.when(g + NB5 < NCH)
            def _next_idx():
                issue_idx(g + NB5, b)
        return 0

    lax.fori_loop(0, NCH // NB5, outer, 0)
    wait_out(NCH - 2, (NCH - 2) % NB5)
    wait_out(NCH - 1, (NCH - 1) % NB5)

    plsc.subcore_barrier()

    @pl.when(sid < WT)
    def _wb_acc():
        pltpu.sync_copy(acc_sh.at[pl.ds(sid * WR, WR)],
                        acc_out.at[cid, pl.ds(sid * WR, WR)])

    @pl.when(cid == 0)
    def _wb_den0():
        pltpu.sync_copy(den_sh.at[pl.ds(sid * TD, TD)],
                        den0_out.at[pl.ds(sid * TD, TD)])

    @pl.when(cid == 1)
    def _wb_den1():
        pltpu.sync_copy(den_sh.at[pl.ds(sid * TD, TD)],
                        den1_out.at[pl.ds(sid * TD, TD)])


_edge_sc = functools.partial(
    pl.kernel,
    out_type=(jax.ShapeDtypeStruct((2, N, HID), jnp.float32),
              jax.ShapeDtypeStruct((NDP,), jnp.float32),
              jax.ShapeDtypeStruct((NDP,), jnp.float32)),
    mesh=_SC_MESH,
    scratch_types=[
        pltpu.VMEM((NB5, C), jnp.int32),
        pltpu.VMEM((NB5, C), jnp.int32),
        pltpu.VMEM((NB5, C), jnp.int32),
        pltpu.VMEM((NB5, C), jnp.int32),
        pltpu.VMEM((NB5, C), jnp.float32),
        pltpu.VMEM((NB5, C), jnp.float32),
        pltpu.VMEM((NB5, C), jnp.float32),
        pltpu.VMEM((NB5, C, HID), jnp.float32),
        pltpu.VMEM((TD,), jnp.float32),
        pltpu.VMEM_SHARED((ACCP, HID), jnp.float32),
        pltpu.VMEM_SHARED((NDP,), jnp.float32),
        pltpu.SemaphoreType.DMA((NB5,)),
        pltpu.SemaphoreType.DMA((NB5,)),
        pltpu.SemaphoreType.DMA((NB5,)),
    ],
)(_edge_sc_body)


def _combine_body(a_ref, d0_ref, d1_ref, b_ref, o_ref):
    den = d0_ref[...] + d1_ref[...]
    o_ref[...] = (a_ref[0] + a_ref[1]) / (den + 1e-16) + b_ref[...]


def _combine(acc, den0, den1, bias, blk=1000):
    return pl.pallas_call(
        _combine_body,
        grid=(N // blk,),
        in_specs=[
            pl.BlockSpec((2, blk, HID), lambda i: (0, i, 0)),
            pl.BlockSpec((blk, 1), lambda i: (i, 0)),
            pl.BlockSpec((blk, 1), lambda i: (i, 0)),
            pl.BlockSpec((1, HID), lambda i: (0, 0)),
        ],
        out_specs=pl.BlockSpec((blk, HID), lambda i: (i, 0)),
        out_shape=jax.ShapeDtypeStruct((N, HID), jnp.float32),
    )(acc, den0[:N].reshape(N, 1), den1[:N].reshape(N, 1),
      bias.reshape(1, HID))


def _edge_phase(xw_flat, tabT, eidx_p, pidx_p, bias):
    qi_flat = tabT[:R].reshape(R * N)
    kj_flat = tabT[R:].reshape(R * N)
    acc, den0, den1 = _edge_sc(xw_flat, qi_flat, kj_flat, eidx_p, pidx_p)
    return _combine(acc, den0, den1, bias)


def kernel(feat_paper, feat_author, lin_w0, lin_b0, ln_g0, ln_b0, lin_w1,
           lin_b1, ln_g1, ln_b1, basis1, comp1, att_q1, att_k1, bias1,
           basis2, comp2, att_q2, att_k2, bias2, tr_g, tr_b, tr_w, tr_bias,
           edge_index, edge_type, forward_type):
    h0 = _prolog(feat_paper, lin_w0, lin_b0, ln_g0, ln_b0, 1000)
    h1 = _prolog(feat_author, lin_w1, lin_b1, ln_g1, ln_b1, 1000)
    x = jnp.concatenate([h0, h1], axis=0)

    src = edge_index[0]
    dst = edge_index[1]
    eidx = edge_type * N + src
    didx = edge_type * N + dst
    pad_g = jnp.broadcast_to(jnp.arange(PADW, dtype=jnp.int32), (NW, PADW))
    pad_d = jnp.broadcast_to(
        N + jnp.arange(PADW, dtype=jnp.int32) % (ACCP - N), (NW, PADW))
    eidx_p = jnp.concatenate([eidx.reshape(NW, EW), pad_g], 1).reshape(-1)
    didx_p = jnp.concatenate([didx.reshape(NW, EW), pad_g], 1).reshape(-1)
    dst_p = jnp.concatenate([dst.reshape(NW, EW), pad_d], 1).reshape(-1)
    pidx_p = didx_p | (dst_p << 17)

    w1 = jnp.einsum('rb,bio->rio', comp1, basis1)
    qk1 = jnp.concatenate([jnp.einsum('rio,o->ri', w1, att_q1[:, 0]),
                           jnp.einsum('rio,o->ri', w1, att_k1[:, 0])], 0)
    w2 = jnp.einsum('rb,bio->rio', comp2, basis2)
    qk2 = jnp.concatenate([jnp.einsum('rio,o->ri', w2, att_q2[:, 0]),
                           jnp.einsum('rio,o->ri', w2, att_k2[:, 0])], 0)

    xw1 = _xw_tables(x, comp1, basis1)
    x = _edge_phase(xw1, _tabs(qk1, x), eidx_p, pidx_p, bias1)

    xw2 = _xw_tables(x, comp2, basis2)
    x = _edge_phase(xw2, _tabs(qk2, x), eidx_p, pidx_p, bias2)

    return _epilog(x, tr_g, tr_b, tr_w, tr_bias)


# speedup vs baseline: 70.6943x; 1.0294x over previous
import functools

import jax
import jax.numpy as jnp
from jax import lax
from jax.experimental import pallas as pl
from jax.experimental.pallas import tpu as pltpu
from jax.experimental.pallas import tpu_sc as plsc

N1, N2 = 6000, 4000
N = N1 + N2
E = 320000
IN, HID, OUT = 128, 128, 64
R, NB = 8, 30

NW = 32
EW = E // NW
C = 64
PADW = 240
EWP = EW + PADW
NCH = EWP // C
NB5 = 5
WT = 10
WR = N // WT
ACCP = 10048
NDP = 10240
TD = NDP // 16
ZR = 40


def _prolog_body(f_ref, w_ref, b_ref, g_ref, bb_ref, o_ref):
    h = jnp.dot(f_ref[...], w_ref[...], preferred_element_type=jnp.float32)
    h = jnp.maximum(h + b_ref[...], 0.0)
    m = jnp.mean(h, axis=-1, keepdims=True)
    v = jnp.mean((h - m) ** 2, axis=-1, keepdims=True)
    o_ref[...] = (h - m) / jnp.sqrt(v + 1e-5) * g_ref[...] + bb_ref[...]


def _prolog(feat, w, b, g, bb, blk):
    n = feat.shape[0]
    return pl.pallas_call(
        _prolog_body,
        grid=(n // blk,),
        in_specs=[
            pl.BlockSpec((blk, IN), lambda i: (i, 0)),
            pl.BlockSpec((IN, IN), lambda i: (0, 0)),
            pl.BlockSpec((1, IN), lambda i: (0, 0)),
            pl.BlockSpec((1, IN), lambda i: (0, 0)),
            pl.BlockSpec((1, IN), lambda i: (0, 0)),
        ],
        out_specs=pl.BlockSpec((blk, IN), lambda i: (i, 0)),
        out_shape=jax.ShapeDtypeStruct((n, IN), jnp.float32),
    )(feat, w, b.reshape(1, IN), g.reshape(1, IN), bb.reshape(1, IN))


def _xw_body(comp_ref, basis_ref, x_ref, xw_ref):
    r = pl.program_id(0)
    comp_row = comp_ref[pl.ds(r, 1), :]
    wr = jnp.dot(comp_row, basis_ref[...].reshape(NB, HID * HID),
                 preferred_element_type=jnp.float32).reshape(HID, HID)
    xw_ref[...] = jnp.dot(x_ref[...], wr, preferred_element_type=jnp.float32)


def _xw_tables(x, comp, basis):
    return pl.pallas_call(
        _xw_body,
        grid=(R,),
        in_specs=[
            pl.BlockSpec((R, NB), lambda r: (0, 0)),
            pl.BlockSpec((NB, HID, HID), lambda r: (0, 0, 0)),
            pl.BlockSpec((N, HID), lambda r: (0, 0)),
        ],
        out_specs=pl.BlockSpec((N, HID), lambda r: (r, 0)),
        out_shape=jax.ShapeDtypeStruct((R * N, HID), jnp.float32),
    )(comp, basis, x)


def _tabs_body(qk_ref, x_ref, o_ref):
    o_ref[...] = lax.dot_general(
        qk_ref[...], x_ref[...], (((1,), (1,)), ((), ())),
        preferred_element_type=jnp.float32)


def _tabs(qkmat, x):
    return pl.pallas_call(
        _tabs_body,
        grid=(1,),
        in_specs=[
            pl.BlockSpec((2 * R, HID), lambda i: (0, 0)),
            pl.BlockSpec((N, HID), lambda i: (0, 0)),
        ],
        out_specs=pl.BlockSpec((2 * R, N), lambda i: (0, 0)),
        out_shape=jax.ShapeDtypeStruct((2 * R, N), jnp.float32),
    )(qkmat, x)


def _epilog_body(x_ref, g_ref, b_ref, w_ref, bias_ref, o_ref):
    x = x_ref[...]
    m = jnp.mean(x, axis=-1, keepdims=True)
    v = jnp.mean((x - m) ** 2, axis=-1, keepdims=True)
    xn = (x - m) / jnp.sqrt(v + 1e-5) * g_ref[...] + b_ref[...]
    o_ref[...] = jnp.dot(xn, w_ref[...], preferred_element_type=jnp.float32) + bias_ref[...]


def _epilog(x, g, b, w, bias):
    return pl.pallas_call(
        _epilog_body,
        grid=(1,),
        in_specs=[
            pl.BlockSpec((N, HID), lambda i: (0, 0)),
            pl.BlockSpec((1, HID), lambda i: (0, 0)),
            pl.BlockSpec((1, HID), lambda i: (0, 0)),
            pl.BlockSpec((HID, OUT), lambda i: (0, 0)),
            pl.BlockSpec((1, OUT), lambda i: (0, 0)),
        ],
        out_specs=pl.BlockSpec((N, OUT), lambda i: (0, 0)),
        out_shape=jax.ShapeDtypeStruct((N, OUT), jnp.float32),
    )(x, g.reshape(1, HID), b.reshape(1, HID), w, bias.reshape(1, OUT))


_SC_MESH = plsc.VectorSubcoreMesh(core_axis_name="c", subcore_axis_name="s")


def _edge_sc_body(xw_hbm, qi_hbm, kj_hbm, eidx_hbm, pidx_hbm,
                  acc_out, den0_out, den1_out,
                  pidx5, eidx5, didx5, dst5, qi5, kj5, ex5, rows_g,
                  zden, acc_sh, den_sh, sem_idx, sem_dat, sem_out):
    cid = lax.axis_index("c")
    sid = lax.axis_index("s")
    wid = sid * 2 + cid

    z16 = jnp.zeros((16,), jnp.float32)

    def zb(i, _):
        for j in range(HID // 16):
            rows_g[0, i, pl.ds(j * 16, 16)] = z16
        return 0

    lax.fori_loop(0, ZR, zb, 0)
    for j in range(TD // 16):
        zden[pl.ds(j * 16, 16)] = z16

    @pl.when(sid < WT)
    def _zero_acc():
        for t in range(WR // ZR):
            pltpu.sync_copy(rows_g.at[0, pl.ds(0, ZR)],
                            acc_sh.at[pl.ds(sid * WR + t * ZR, ZR)])

    pltpu.sync_copy(zden, den_sh.at[pl.ds(sid * TD, TD)])
    plsc.subcore_barrier()

    def issue_idx(g, b):
        base = wid * EWP + g * C
        pltpu.async_copy(eidx_hbm.at[pl.ds(base, C)], eidx5.at[b],
                         sem_idx.at[b])
        pltpu.async_copy(pidx_hbm.at[pl.ds(base, C)], pidx5.at[b],
                         sem_idx.at[b])

    def wait_idx(g, b):
        base = wid * EWP + g * C
        pltpu.make_async_copy(eidx_hbm.at[pl.ds(base, C)], eidx5.at[b],
                              sem_idx.at[b]).wait()
        pltpu.make_async_copy(pidx_hbm.at[pl.ds(base, C)], pidx5.at[b],
                              sem_idx.at[b]).wait()

    def unpack_idx(b):
        for j in range(C // 16):
            sl = pl.ds(j * 16, 16)
            v = pidx5[b, sl]
            didx5[b, sl] = v & 0x1FFFF
            dst5[b, sl] = v >> 17

    def issue_dat(b):
        pltpu.async_copy(qi_hbm.at[didx5.at[b]], qi5.at[b], sem_dat.at[b])
        pltpu.async_copy(kj_hbm.at[eidx5.at[b]], kj5.at[b], sem_dat.at[b])
        pltpu.async_copy(xw_hbm.at[eidx5.at[b]], rows_g.at[b], sem_dat.at[b])

    def wait_qk(b):
        pltpu.make_async_copy(qi_hbm.at[didx5.at[b]], qi5.at[b],
                              sem_dat.at[b]).wait()
        pltpu.make_async_copy(kj_hbm.at[eidx5.at[b]], kj5.at[b],
                              sem_dat.at[b]).wait()

    def wait_rows(b):
        pltpu.make_async_copy(xw_hbm.at[eidx5.at[b]], rows_g.at[b],
                              sem_dat.at[b]).wait()

    def issue_out(g, b):
        pltpu.async_copy(rows_g.at[b], acc_sh.at[dst5.at[b]],
                         sem_out.at[b], add=True)
        pltpu.async_copy(ex5.at[b], den_sh.at[dst5.at[b]],
                         sem_out.at[b], add=True)

    def wait_out(g, b):
        pltpu.make_async_copy(rows_g.at[b], acc_sh.at[dst5.at[b]],
                              sem_out.at[b]).wait()
        pltpu.make_async_copy(ex5.at[b], den_sh.at[dst5.at[b]],
                              sem_out.at[b]).wait()

    for gg in range(NB5):
        issue_idx(gg, gg)
    for gg in range(3):
        wait_idx(gg, gg)
        unpack_idx(gg)
        issue_dat(gg)

    def outer(o, _):
        for b in range(NB5):
            g = o * NB5 + b
            b3 = (b + 3) % NB5

            @pl.when(g >= 2)
            def _drain():
                wait_out(g - 2, b3)

            wait_qk(b)
            for j in range(C // 16):
                sl = pl.ds(j * 16, 16)
                a = qi5[b, sl] + kj5[b, sl]
                a = jnp.where(a >= 0.0, a, 0.2 * a)
                ex5[b, sl] = jnp.exp(a)
            @pl.when(g + 3 < NCH)
            def _next_dat():
                wait_idx(g + 3, b3)
                unpack_idx(b3)
                issue_dat(b3)

            wait_rows(b)

            def scale_grp(g2, _, b=b):
                ex16 = ex5[b, pl.ds(g2 * 16, 16)]
                for l in range(16):
                    s = ex16[l]
                    row = g2 * 16 + l
                    for j in range(HID // 16):
                        sl = pl.ds(j * 16, 16)
                        rows_g[b, row, sl] = rows_g[b, row, sl] * s
                return 0

            lax.fori_loop(0, C // 16, scale_grp, 0)
            issue_out(g, b)

            @pl.when(g + NB5 < NCH)
            def _next_idx():
                issue_idx(g + NB5, b)
        return 0

    lax.fori_loop(0, NCH // NB5, outer, 0)
    wait_out(NCH - 2, (NCH - 2) % NB5)
    wait_out(NCH - 1, (NCH - 1) % NB5)

    plsc.subcore_barrier()

    @pl.when(sid < WT)
    def _wb_acc():
        pltpu.sync_copy(acc_sh.at[pl.ds(sid * WR, WR)],
                        acc_out.at[cid, pl.ds(sid * WR, WR)])

    @pl.when(cid == 0)
    def _wb_den0():
        pltpu.sync_copy(den_sh.at[pl.ds(sid * TD, TD)],
                        den0_out.at[pl.ds(sid * TD, TD)])

    @pl.when(cid == 1)
    def _wb_den1():
        pltpu.sync_copy(den_sh.at[pl.ds(sid * TD, TD)],
                        den1_out.at[pl.ds(sid * TD, TD)])


_edge_sc = functools.partial(
    pl.kernel,
    out_type=(jax.ShapeDtypeStruct((2, N, HID), jnp.float32),
              jax.ShapeDtypeStruct((NDP,), jnp.float32),
              jax.ShapeDtypeStruct((NDP,), jnp.float32)),
    mesh=_SC_MESH,
    scratch_types=[
        pltpu.VMEM((NB5, C), jnp.int32),
        pltpu.VMEM((NB5, C), jnp.int32),
        pltpu.VMEM((NB5, C), jnp.int32),
        pltpu.VMEM((NB5, C), jnp.int32),
        pltpu.VMEM((NB5, C), jnp.float32),
        pltpu.VMEM((NB5, C), jnp.float32),
        pltpu.VMEM((NB5, C), jnp.float32),
        pltpu.VMEM((NB5, C, HID), jnp.float32),
        pltpu.VMEM((TD,), jnp.float32),
        pltpu.VMEM_SHARED((ACCP, HID), jnp.float32),
        pltpu.VMEM_SHARED((NDP,), jnp.float32),
        pltpu.SemaphoreType.DMA((NB5,)),
        pltpu.SemaphoreType.DMA((NB5,)),
        pltpu.SemaphoreType.DMA((NB5,)),
    ],
)(_edge_sc_body)


def _combine_body(a_ref, r_ref, b_ref, o_ref):
    o_ref[...] = (a_ref[0] + a_ref[1]) * r_ref[...] + b_ref[...]


def _combine(acc, den0, den1, bias, blk=1000):
    recip = 1.0 / (den0[:N] + den1[:N] + 1e-16)
    recipb = jnp.broadcast_to(recip[:, None], (N, HID))
    return pl.pallas_call(
        _combine_body,
        grid=(N // blk,),
        in_specs=[
            pl.BlockSpec((2, blk, HID), lambda i: (0, i, 0)),
            pl.BlockSpec((blk, HID), lambda i: (i, 0)),
            pl.BlockSpec((1, HID), lambda i: (0, 0)),
        ],
        out_specs=pl.BlockSpec((blk, HID), lambda i: (i, 0)),
        out_shape=jax.ShapeDtypeStruct((N, HID), jnp.float32),
    )(acc, recipb, bias.reshape(1, HID))


def _edge_phase(xw_flat, tabT, eidx_p, pidx_p, bias):
    qi_flat = tabT[:R].reshape(R * N)
    kj_flat = tabT[R:].reshape(R * N)
    acc, den0, den1 = _edge_sc(xw_flat, qi_flat, kj_flat, eidx_p, pidx_p)
    return _combine(acc, den0, den1, bias)


def kernel(feat_paper, feat_author, lin_w0, lin_b0, ln_g0, ln_b0, lin_w1,
           lin_b1, ln_g1, ln_b1, basis1, comp1, att_q1, att_k1, bias1,
           basis2, comp2, att_q2, att_k2, bias2, tr_g, tr_b, tr_w, tr_bias,
           edge_index, edge_type, forward_type):
    h0 = _prolog(feat_paper, lin_w0, lin_b0, ln_g0, ln_b0, 1000)
    h1 = _prolog(feat_author, lin_w1, lin_b1, ln_g1, ln_b1, 1000)
    x = jnp.concatenate([h0, h1], axis=0)

    src = edge_index[0]
    dst = edge_index[1]
    eidx = edge_type * N + src
    didx = edge_type * N + dst
    pad_g = jnp.broadcast_to(jnp.arange(PADW, dtype=jnp.int32), (NW, PADW))
    pad_d = jnp.broadcast_to(
        N + jnp.arange(PADW, dtype=jnp.int32) % (ACCP - N), (NW, PADW))
    eidx_p = jnp.concatenate([eidx.reshape(NW, EW), pad_g], 1).reshape(-1)
    didx_p = jnp.concatenate([didx.reshape(NW, EW), pad_g], 1).reshape(-1)
    dst_p = jnp.concatenate([dst.reshape(NW, EW), pad_d], 1).reshape(-1)
    pidx_p = didx_p | (dst_p << 17)

    w1 = jnp.einsum('rb,bio->rio', comp1, basis1)
    qk1 = jnp.concatenate([jnp.einsum('rio,o->ri', w1, att_q1[:, 0]),
                           jnp.einsum('rio,o->ri', w1, att_k1[:, 0])], 0)
    w2 = jnp.einsum('rb,bio->rio', comp2, basis2)
    qk2 = jnp.concatenate([jnp.einsum('rio,o->ri', w2, att_q2[:, 0]),
                           jnp.einsum('rio,o->ri', w2, att_k2[:, 0])], 0)

    xw1 = _xw_tables(x, comp1, basis1)
    x = _edge_phase(xw1, _tabs(qk1, x), eidx_p, pidx_p, bias1)

    xw2 = _xw_tables(x, comp2, basis2)
    x = _edge_phase(xw2, _tabs(qk2, x), eidx_p, pidx_p, bias2)

    return _epilog(x, tr_g, tr_b, tr_w, tr_bias)


# speedup vs baseline: 71.1337x; 1.0062x over previous
import functools

import jax
import jax.numpy as jnp
from jax import lax
from jax.experimental import pallas as pl
from jax.experimental.pallas import tpu as pltpu
from jax.experimental.pallas import tpu_sc as plsc

N1, N2 = 6000, 4000
N = N1 + N2
E = 320000
IN, HID, OUT = 128, 128, 64
R, NB = 8, 30

NW = 32
EW = E // NW
C = 64
PADW = 240
EWP = EW + PADW
NCH = EWP // C
NB5 = 5
WT = 10
WR = N // WT
ACCP = 10048
NDP = 10240
TD = NDP // 16
ZR = 40


def _prolog_body(f_ref, w_ref, b_ref, g_ref, bb_ref, o_ref):
    h = jnp.dot(f_ref[...], w_ref[...], preferred_element_type=jnp.float32)
    h = jnp.maximum(h + b_ref[...], 0.0)
    m = jnp.mean(h, axis=-1, keepdims=True)
    v = jnp.mean((h - m) ** 2, axis=-1, keepdims=True)
    o_ref[...] = (h - m) / jnp.sqrt(v + 1e-5) * g_ref[...] + bb_ref[...]


def _prolog(feat, w, b, g, bb, blk):
    n = feat.shape[0]
    return pl.pallas_call(
        _prolog_body,
        grid=(n // blk,),
        in_specs=[
            pl.BlockSpec((blk, IN), lambda i: (i, 0)),
            pl.BlockSpec((IN, IN), lambda i: (0, 0)),
            pl.BlockSpec((1, IN), lambda i: (0, 0)),
            pl.BlockSpec((1, IN), lambda i: (0, 0)),
            pl.BlockSpec((1, IN), lambda i: (0, 0)),
        ],
        out_specs=pl.BlockSpec((blk, IN), lambda i: (i, 0)),
        out_shape=jax.ShapeDtypeStruct((n, IN), jnp.float32),
    )(feat, w, b.reshape(1, IN), g.reshape(1, IN), bb.reshape(1, IN))


def _xw_body(w_ref, x_ref, xw_ref):
    xw_ref[...] = jnp.dot(x_ref[...], w_ref[0],
                          preferred_element_type=jnp.float32)


def _xw_tables(x, w):
    return pl.pallas_call(
        _xw_body,
        grid=(R,),
        in_specs=[
            pl.BlockSpec((1, HID, HID), lambda r: (r, 0, 0)),
            pl.BlockSpec((N, HID), lambda r: (0, 0)),
        ],
        out_specs=pl.BlockSpec((N, HID), lambda r: (r, 0)),
        out_shape=jax.ShapeDtypeStruct((R * N, HID), jnp.float32),
    )(w, x)


def _tabs_body(qk_ref, x_ref, o_ref):
    o_ref[...] = lax.dot_general(
        qk_ref[...], x_ref[...], (((1,), (1,)), ((), ())),
        preferred_element_type=jnp.float32)


def _tabs(qkmat, x):
    return pl.pallas_call(
        _tabs_body,
        grid=(1,),
        in_specs=[
            pl.BlockSpec((2 * R, HID), lambda i: (0, 0)),
            pl.BlockSpec((N, HID), lambda i: (0, 0)),
        ],
        out_specs=pl.BlockSpec((2 * R, N), lambda i: (0, 0)),
        out_shape=jax.ShapeDtypeStruct((2 * R, N), jnp.float32),
    )(qkmat, x)


def _epilog_body(x_ref, g_ref, b_ref, w_ref, bias_ref, o_ref):
    x = x_ref[...]
    m = jnp.mean(x, axis=-1, keepdims=True)
    v = jnp.mean((x - m) ** 2, axis=-1, keepdims=True)
    xn = (x - m) / jnp.sqrt(v + 1e-5) * g_ref[...] + b_ref[...]
    o_ref[...] = jnp.dot(xn, w_ref[...], preferred_element_type=jnp.float32) + bias_ref[...]


def _epilog(x, g, b, w, bias):
    return pl.pallas_call(
        _epilog_body,
        grid=(1,),
        in_specs=[
            pl.BlockSpec((N, HID), lambda i: (0, 0)),
            pl.BlockSpec((1, HID), lambda i: (0, 0)),
            pl.BlockSpec((1, HID), lambda i: (0, 0)),
            pl.BlockSpec((HID, OUT), lambda i: (0, 0)),
            pl.BlockSpec((1, OUT), lambda i: (0, 0)),
        ],
        out_specs=pl.BlockSpec((N, OUT), lambda i: (0, 0)),
        out_shape=jax.ShapeDtypeStruct((N, OUT), jnp.float32),
    )(x, g.reshape(1, HID), b.reshape(1, HID), w, bias.reshape(1, OUT))


_SC_MESH = plsc.VectorSubcoreMesh(core_axis_name="c", subcore_axis_name="s")


def _edge_sc_body(xw_hbm, qi_hbm, kj_hbm, eidx_hbm, pidx_hbm,
                  acc_out, den0_out, den1_out,
                  pidx5, eidx5, didx5, dst5, qi5, kj5, ex5, rows_g,
                  zden, acc_sh, den_sh, sem_idx, sem_dat, sem_out):
    cid = lax.axis_index("c")
    sid = lax.axis_index("s")
    wid = sid * 2 + cid

    z16 = jnp.zeros((16,), jnp.float32)

    def zb(i, _):
        for j in range(HID // 16):
            rows_g[0, i, pl.ds(j * 16, 16)] = z16
        return 0

    lax.fori_loop(0, ZR, zb, 0)
    for j in range(TD // 16):
        zden[pl.ds(j * 16, 16)] = z16

    @pl.when(sid < WT)
    def _zero_acc():
        for t in range(WR // ZR):
            pltpu.sync_copy(rows_g.at[0, pl.ds(0, ZR)],
                            acc_sh.at[pl.ds(sid * WR + t * ZR, ZR)])

    pltpu.sync_copy(zden, den_sh.at[pl.ds(sid * TD, TD)])
    plsc.subcore_barrier()

    def issue_idx(g, b):
        base = wid * EWP + g * C
        pltpu.async_copy(eidx_hbm.at[pl.ds(base, C)], eidx5.at[b],
                         sem_idx.at[b])
        pltpu.async_copy(pidx_hbm.at[pl.ds(base, C)], pidx5.at[b],
                         sem_idx.at[b])

    def wait_idx(g, b):
        base = wid * EWP + g * C
        pltpu.make_async_copy(eidx_hbm.at[pl.ds(base, C)], eidx5.at[b],
                              sem_idx.at[b]).wait()
        pltpu.make_async_copy(pidx_hbm.at[pl.ds(base, C)], pidx5.at[b],
                              sem_idx.at[b]).wait()

    def unpack_idx(b):
        for j in range(C // 16):
            sl = pl.ds(j * 16, 16)
            v = pidx5[b, sl]
            didx5[b, sl] = v & 0x1FFFF
            dst5[b, sl] = v >> 17

    def issue_dat(b):
        pltpu.async_copy(qi_hbm.at[didx5.at[b]], qi5.at[b], sem_dat.at[b])
        pltpu.async_copy(kj_hbm.at[eidx5.at[b]], kj5.at[b], sem_dat.at[b])
        pltpu.async_copy(xw_hbm.at[eidx5.at[b]], rows_g.at[b], sem_dat.at[b])

    def wait_qk(b):
        pltpu.make_async_copy(qi_hbm.at[didx5.at[b]], qi5.at[b],
                              sem_dat.at[b]).wait()
        pltpu.make_async_copy(kj_hbm.at[eidx5.at[b]], kj5.at[b],
                              sem_dat.at[b]).wait()

    def wait_rows(b):
        pltpu.make_async_copy(xw_hbm.at[eidx5.at[b]], rows_g.at[b],
                              sem_dat.at[b]).wait()

    def issue_out(g, b):
        pltpu.async_copy(rows_g.at[b], acc_sh.at[dst5.at[b]],
                         sem_out.at[b], add=True)
        pltpu.async_copy(ex5.at[b], den_sh.at[dst5.at[b]],
                         sem_out.at[b], add=True)

    def wait_out(g, b):
        pltpu.make_async_copy(rows_g.at[b], acc_sh.at[dst5.at[b]],
                              sem_out.at[b]).wait()
        pltpu.make_async_copy(ex5.at[b], den_sh.at[dst5.at[b]],
                              sem_out.at[b]).wait()

    for gg in range(NB5):
        issue_idx(gg, gg)
    for gg in range(3):
        wait_idx(gg, gg)
        unpack_idx(gg)
        issue_dat(gg)

    def outer(o, _):
        for b in range(NB5):
            g = o * NB5 + b
            b3 = (b + 3) % NB5

            @pl.when(g >= 2)
            def _drain():
                wait_out(g - 2, b3)

            wait_qk(b)
            for j in range(C // 16):
                sl = pl.ds(j * 16, 16)
                a = qi5[b, sl] + kj5[b, sl]
                a = jnp.where(a >= 0.0, a, 0.2 * a)
                ex5[b, sl] = jnp.exp(a)
            @pl.when(g + 3 < NCH)
            def _next_dat():
                wait_idx(g + 3, b3)
                unpack_idx(b3)
                issue_dat(b3)

            wait_rows(b)

            def scale_grp(g2, _, b=b):
                ex16 = ex5[b, pl.ds(g2 * 16, 16)]
                for l in range(16):
                    s = ex16[l]
                    row = g2 * 16 + l
                    for j in range(HID // 16):
                        sl = pl.ds(j * 16, 16)
                        rows_g[b, row, sl] = rows_g[b, row, sl] * s
                return 0

            lax.fori_loop(0, C // 16, scale_grp, 0)
            issue_out(g, b)

            @pl.when(g + NB5 < NCH)
            def _next_idx():
                issue_idx(g + NB5, b)
        return 0

    lax.fori_loop(0, NCH // NB5, outer, 0)
    wait_out(NCH - 2, (NCH - 2) % NB5)
    wait_out(NCH - 1, (NCH - 1) % NB5)

    plsc.subcore_barrier()

    @pl.when(sid < WT)
    def _wb_acc():
        pltpu.sync_copy(acc_sh.at[pl.ds(sid * WR, WR)],
                        acc_out.at[cid, pl.ds(sid * WR, WR)])

    @pl.when(cid == 0)
    def _wb_den0():
        pltpu.sync_copy(den_sh.at[pl.ds(sid * TD, TD)],
                        den0_out.at[pl.ds(sid * TD, TD)])

    @pl.when(cid == 1)
    def _wb_den1():
        pltpu.sync_copy(den_sh.at[pl.ds(sid * TD, TD)],
                        den1_out.at[pl.ds(sid * TD, TD)])


_edge_sc = functools.partial(
    pl.kernel,
    out_type=(jax.ShapeDtypeStruct((2, N, HID), jnp.float32),
              jax.ShapeDtypeStruct((NDP,), jnp.float32),
              jax.ShapeDtypeStruct((NDP,), jnp.float32)),
    mesh=_SC_MESH,
    scratch_types=[
        pltpu.VMEM((NB5, C), jnp.int32),
        pltpu.VMEM((NB5, C), jnp.int32),
        pltpu.VMEM((NB5, C), jnp.int32),
        pltpu.VMEM((NB5, C), jnp.int32),
        pltpu.VMEM((NB5, C), jnp.float32),
        pltpu.VMEM((NB5, C), jnp.float32),
        pltpu.VMEM((NB5, C), jnp.float32),
        pltpu.VMEM((NB5, C, HID), jnp.float32),
        pltpu.VMEM((TD,), jnp.float32),
        pltpu.VMEM_SHARED((ACCP, HID), jnp.float32),
        pltpu.VMEM_SHARED((NDP,), jnp.float32),
        pltpu.SemaphoreType.DMA((NB5,)),
        pltpu.SemaphoreType.DMA((NB5,)),
        pltpu.SemaphoreType.DMA((NB5,)),
    ],
)(_edge_sc_body)


def _combine_body(a_ref, r_ref, b_ref, o_ref):
    o_ref[...] = (a_ref[0] + a_ref[1]) * r_ref[...] + b_ref[...]


def _combine(acc, den0, den1, bias, blk=2000):
    recip = 1.0 / (den0[:N] + den1[:N] + 1e-16)
    recipb = jnp.broadcast_to(recip[:, None], (N, HID))
    return pl.pallas_call(
        _combine_body,
        grid=(N // blk,),
        in_specs=[
            pl.BlockSpec((2, blk, HID), lambda i: (0, i, 0)),
            pl.BlockSpec((blk, HID), lambda i: (i, 0)),
            pl.BlockSpec((1, HID), lambda i: (0, 0)),
        ],
        out_specs=pl.BlockSpec((blk, HID), lambda i: (i, 0)),
        out_shape=jax.ShapeDtypeStruct((N, HID), jnp.float32),
    )(acc, recipb, bias.reshape(1, HID))


def _edge_phase(xw_flat, tabT, eidx_p, pidx_p, bias):
    qi_flat = tabT[:R].reshape(R * N)
    kj_flat = tabT[R:].reshape(R * N)
    acc, den0, den1 = _edge_sc(xw_flat, qi_flat, kj_flat, eidx_p, pidx_p)
    return _combine(acc, den0, den1, bias)


def kernel(feat_paper, feat_author, lin_w0, lin_b0, ln_g0, ln_b0, lin_w1,
           lin_b1, ln_g1, ln_b1, basis1, comp1, att_q1, att_k1, bias1,
           basis2, comp2, att_q2, att_k2, bias2, tr_g, tr_b, tr_w, tr_bias,
           edge_index, edge_type, forward_type):
    h0 = _prolog(feat_paper, lin_w0, lin_b0, ln_g0, ln_b0, 2000)
    h1 = _prolog(feat_author, lin_w1, lin_b1, ln_g1, ln_b1, 2000)
    x = jnp.concatenate([h0, h1], axis=0)

    src = edge_index[0]
    dst = edge_index[1]
    eidx = edge_type * N + src
    didx = edge_type * N + dst
    pad_g = jnp.broadcast_to(jnp.arange(PADW, dtype=jnp.int32), (NW, PADW))
    pad_d = jnp.broadcast_to(
        N + jnp.arange(PADW, dtype=jnp.int32) % (ACCP - N), (NW, PADW))
    eidx_p = jnp.concatenate([eidx.reshape(NW, EW), pad_g], 1).reshape(-1)
    didx_p = jnp.concatenate([didx.reshape(NW, EW), pad_g], 1).reshape(-1)
    dst_p = jnp.concatenate([dst.reshape(NW, EW), pad_d], 1).reshape(-1)
    pidx_p = didx_p | (dst_p << 17)

    w1 = jnp.einsum('rb,bio->rio', comp1, basis1)
    qk1 = jnp.concatenate([jnp.einsum('rio,o->ri', w1, att_q1[:, 0]),
                           jnp.einsum('rio,o->ri', w1, att_k1[:, 0])], 0)
    w2 = jnp.einsum('rb,bio->rio', comp2, basis2)
    qk2 = jnp.concatenate([jnp.einsum('rio,o->ri', w2, att_q2[:, 0]),
                           jnp.einsum('rio,o->ri', w2, att_k2[:, 0])], 0)

    xw1 = _xw_tables(x, w1)
    x = _edge_phase(xw1, _tabs(qk1, x), eidx_p, pidx_p, bias1)

    xw2 = _xw_tables(x, w2)
    x = _edge_phase(xw2, _tabs(qk2, x), eidx_p, pidx_p, bias2)

    return _epilog(x, tr_g, tr_b, tr_w, tr_bias)
